# Initial kernel scaffold; baseline (speedup 1.0000x reference)
import jax
import jax.numpy as jnp
from jax.experimental import pallas as pl


def kernel(x, label, idx, emb):
    raise NotImplementedError("write your pallas kernel here")


# speedup vs baseline: 1.3590x; 1.3590x over previous
import functools

import jax
import jax.numpy as jnp
from jax.experimental import pallas as pl

N_E = 1024
E_DIM = 64
MU = 0.25
BM = 1024


def _vq_block(x_ref, emb_ref, z2_ref, e2_ref, xq_ref, idx_ref, loss_ref):
    z = x_ref[...]
    emb = emb_ref[...]
    z2 = z2_ref[...]
    e2 = e2_ref[...]
    dots = jnp.dot(z, emb.T, preferred_element_type=jnp.float32)
    d = (z2 + e2) - 2.0 * dots
    dmin = jnp.min(d, axis=1, keepdims=True)
    col = jax.lax.broadcasted_iota(jnp.int32, (BM, N_E), 1)
    idxs = jnp.min(jnp.where(d == dmin, col, N_E), axis=1).astype(jnp.int32)
    onehot = (col == idxs[:, None]).astype(jnp.float32)
    xq = jnp.dot(onehot, emb, preferred_element_type=jnp.float32)
    xq_ref[...] = xq
    idx_ref[...] = idxs

    @pl.when(pl.program_id(0) == 0)
    def _():
        loss_ref[...] = jnp.zeros((1, 1), jnp.float32)

    diff = xq - z
    loss_ref[...] += jnp.sum(diff * diff).reshape(1, 1)


@jax.jit
def _vq(latent, emb, z2, e2):
    m = latent.shape[0]
    grid = (m // BM,)
    xq, idxs, loss_sum = pl.pallas_call(
        _vq_block,
        grid=grid,
        in_specs=[
            pl.BlockSpec((BM, E_DIM), lambda i: (i, 0)),
            pl.BlockSpec((N_E, E_DIM), lambda i: (0, 0)),
            pl.BlockSpec((BM, 1), lambda i: (i, 0)),
            pl.BlockSpec((1, N_E), lambda i: (0, 0)),
        ],
        out_specs=[
            pl.BlockSpec((BM, E_DIM), lambda i: (i, 0)),
            pl.BlockSpec((BM,), lambda i: (i,)),
            pl.BlockSpec((1, 1), lambda i: (0, 0)),
        ],
        out_shape=[
            jax.ShapeDtypeStruct((m, E_DIM), jnp.float32),
            jax.ShapeDtypeStruct((m,), jnp.int32),
            jax.ShapeDtypeStruct((1, 1), jnp.float32),
        ],
    )(latent, emb, z2, e2)
    return xq, idxs, loss_sum


def kernel(x, label, idx, emb):
    latent = x.reshape(-1, E_DIM)
    z2 = jnp.sum(latent ** 2, axis=1, keepdims=True)
    e2 = jnp.sum(emb ** 2, axis=1)[None, :]
    xq, idxs, loss_sum = _vq(latent, emb, z2, e2)
    n = latent.shape[0] * E_DIM
    loss = loss_sum[0, 0] * ((1.0 + MU) / n)
    x_q_st = xq.reshape(x.shape)
    indices_out = idxs.reshape(x.shape[:-1])
    return (x_q_st, loss, indices_out)
